# baseline (speedup 1.0000x reference)
import jax
import jax.numpy as jnp
from jax import lax
from jax.experimental import pallas as pl
from jax.experimental.pallas import tpu as pltpu
from jax.experimental.pallas import tpu_sc as plsc

_BIGRAM_VOCAB = 1000000
_BIGRAM_DIM = 64
_MODEL_DIM = 1024
_BATCH = 4
_SEQ = 8192
_NTOK = _BATCH * _SEQ
_NC, _NS, _L = 2, 16, 16
_NW = _NC * _NS
_CHUNK = _NTOK // _NW
_NSLICE = _CHUNK // _L
_GB = 128
_NG = _CHUNK // _GB

_PB = 16384
_HALF = 507904
_NPAIR = _HALF
_LINROWS = 2 * _HALF
_VBLK = -(-_BIGRAM_VOCAB // _PB) - 1


def _pack_body(a_ref, b_ref, o_ref):
    o_ref[...] = jnp.concatenate(
        [a_ref[...].T, b_ref[...].T], axis=1
    ).astype(jnp.bfloat16)


def _pack(emb_t):
    return pl.pallas_call(
        _pack_body,
        grid=(_NPAIR // _PB,),
        in_specs=[
            pl.BlockSpec((_BIGRAM_DIM, _PB), lambda i: (0, i)),
            pl.BlockSpec(
                (_BIGRAM_DIM, _PB),
                lambda i: (0, jnp.minimum(_HALF // _PB + i, _VBLK)),
            ),
        ],
        out_specs=pl.BlockSpec((_PB, 2 * _BIGRAM_DIM), lambda i: (i, 0)),
        out_shape=jax.ShapeDtypeStruct((_NPAIR, 2 * _BIGRAM_DIM), jnp.bfloat16),
    )(emb_t, emb_t)


def _hash_body(tok_hbm, idx_hbm, tok_v, idx_v):
    wid = lax.axis_index("s") * _NC + lax.axis_index("c")
    base = wid * _CHUNK
    row_start = (base % _SEQ) == 0

    @pl.when(row_start)
    def _():
        pltpu.sync_copy(tok_hbm.at[pl.ds(base, _CHUNK)], tok_v.at[pl.ds(_L, _CHUNK)])

    @pl.when(jnp.logical_not(row_start))
    def _():
        pltpu.sync_copy(tok_hbm.at[pl.ds(base - _L, _CHUNK + _L)], tok_v)

    lane = lax.iota(jnp.int32, _L)
    first_lane = jnp.int32(1) - jnp.minimum(lane, jnp.int32(1))
    flag = row_start.astype(jnp.int32)
    for i in range(_NSLICE):
        cur = tok_v[pl.ds(_L + i * _L, _L)]
        prev = tok_v[pl.ds(_L - 1 + i * _L, _L)]
        mixed = jnp.bitwise_xor(jnp.int32(36313) * cur, jnp.int32(27191) * prev)
        h = lax.rem(mixed, jnp.int32(_BIGRAM_VOCAB - 1))
        if i == 0:
            h = h + first_lane * flag * (jnp.int32(_BIGRAM_VOCAB - 1) - h)
        d = h - jnp.int32(_HALF)
        ge = jnp.int32(1) - lax.shift_right_logical(d, jnp.int32(31))
        j = jnp.int32(2) * (h - jnp.int32(_HALF) * ge) + ge
        idx_v[i // (_GB // _L), pl.ds((i % (_GB // _L)) * _L, _L)] = j

    pltpu.sync_copy(idx_v, idx_hbm.at[wid])


def _gather_body(idx_hbm, table_hbm, gat_hbm, idx_v, rows_v, sem):
    wid = lax.axis_index("s") * _NC + lax.axis_index("c")
    base = wid * _CHUNK
    pltpu.sync_copy(idx_hbm.at[wid], idx_v)

    copies = []
    for j in range(_NG):
        cp = pltpu.make_async_copy(
            table_hbm.at[idx_v.at[j]], rows_v.at[pl.ds(j * _GB, _GB)], sem
        )
        cp.start()
        copies.append(cp)
    for cp in copies:
        cp.wait()

    pltpu.sync_copy(rows_v, gat_hbm.at[pl.ds(base, _CHUNK), pl.ds(0, _BIGRAM_DIM)])


_sc_hash = pl.kernel(
    _hash_body,
    out_type=jax.ShapeDtypeStruct((_NW, _NG, _GB), jnp.int32),
    name="sc_hash",
    mesh=plsc.VectorSubcoreMesh(core_axis_name="c", subcore_axis_name="s"),
    scratch_types=[
        pltpu.VMEM((_L + _CHUNK,), jnp.int32),
        pltpu.VMEM((_NG, _GB), jnp.int32),
    ],
    compiler_params=pltpu.CompilerParams(use_tc_tiling_on_sc=False),
)

_sc_gather = pl.kernel(
    _gather_body,
    out_type=jax.ShapeDtypeStruct((_NTOK, 2 * _BIGRAM_DIM), jnp.bfloat16),
    name="sc_gather",
    mesh=plsc.VectorSubcoreMesh(core_axis_name="c", subcore_axis_name="s"),
    scratch_types=[
        pltpu.VMEM((_NG, _GB), jnp.int32),
        pltpu.VMEM((_CHUNK, _BIGRAM_DIM), jnp.bfloat16),
        pltpu.SemaphoreType.DMA,
    ],
    compiler_params=pltpu.CompilerParams(use_tc_tiling_on_sc=False),
)

_RB = 4096


def _mm_body(s_ref, x_ref, w_ref, o_ref):
    xa = x_ref[:, : _BIGRAM_DIM]
    acc = lax.dot_general(
        xa,
        w_ref[...].astype(jnp.bfloat16),
        dimension_numbers=(((1,), (1,)), ((), ())),
        preferred_element_type=jnp.float32,
    )
    o_ref[...] = acc * s_ref[0]


def _tc_proj(x2, w, scale):
    return pl.pallas_call(
        _mm_body,
        grid=(_NTOK // _RB,),
        in_specs=[
            pl.BlockSpec(memory_space=pltpu.SMEM),
            pl.BlockSpec((_RB, 2 * _BIGRAM_DIM), lambda i: (i, 0)),
            pl.BlockSpec((_MODEL_DIM, _BIGRAM_DIM), lambda i: (0, 0)),
        ],
        out_specs=pl.BlockSpec((_RB, _MODEL_DIM), lambda i: (i, 0)),
        out_shape=jax.ShapeDtypeStruct((_NTOK, _MODEL_DIM), jnp.float32),
    )(scale.reshape(1), x2, w)


def kernel(token_ids, embed_weight, proj_weight, scale):
    tok = token_ids.astype(jnp.int32).reshape(_NTOK)
    idx = _sc_hash(tok)
    pairs = _pack(embed_weight.T)
    lin = pairs.reshape(_LINROWS, _BIGRAM_DIM)
    gathered = _sc_gather(idx, lin)
    out = _tc_proj(gathered, proj_weight, scale)
    return out.reshape(_BATCH, _SEQ, _MODEL_DIM)


# speedup vs baseline: 2.4277x; 2.4277x over previous
import jax
import jax.numpy as jnp
from jax import lax
from jax.experimental import pallas as pl
from jax.experimental.pallas import tpu as pltpu
from jax.experimental.pallas import tpu_sc as plsc

_BIGRAM_VOCAB = 1000000
_BIGRAM_DIM = 64
_MODEL_DIM = 1024
_BATCH = 4
_SEQ = 8192
_NTOK = _BATCH * _SEQ
_NC, _NS, _L = 2, 16, 16
_NW = _NC * _NS
_CHUNK = _NTOK // _NW
_NSLICE = _CHUNK // _L
_GB = 128
_NG = _CHUNK // _GB

_PB = 16384
_HALF = 507904
_NPAIR = _HALF
_LINROWS = 2 * _HALF
_VBLK = -(-_BIGRAM_VOCAB // _PB) - 1


def _pack_body(a_ref, b_ref, o_ref):
    o_ref[...] = jnp.concatenate([a_ref[...].T, b_ref[...].T], axis=1)


def _pack(emb_t):
    return pl.pallas_call(
        _pack_body,
        grid=(_NPAIR // _PB,),
        in_specs=[
            pl.BlockSpec((_BIGRAM_DIM, _PB), lambda i: (0, i)),
            pl.BlockSpec(
                (_BIGRAM_DIM, _PB),
                lambda i: (0, jnp.minimum(_HALF // _PB + i, _VBLK)),
            ),
        ],
        out_specs=pl.BlockSpec((_PB, 2 * _BIGRAM_DIM), lambda i: (i, 0)),
        out_shape=jax.ShapeDtypeStruct((_NPAIR, 2 * _BIGRAM_DIM), jnp.float32),
    )(emb_t, emb_t)


def _hash_body(tok_hbm, idx_hbm, tok_v, idx_v):
    wid = lax.axis_index("s") * _NC + lax.axis_index("c")
    base = wid * _CHUNK
    row_start = (base % _SEQ) == 0

    @pl.when(row_start)
    def _():
        pltpu.sync_copy(tok_hbm.at[pl.ds(base, _CHUNK)], tok_v.at[pl.ds(_L, _CHUNK)])

    @pl.when(jnp.logical_not(row_start))
    def _():
        pltpu.sync_copy(tok_hbm.at[pl.ds(base - _L, _CHUNK + _L)], tok_v)

    lane = lax.iota(jnp.int32, _L)
    first_lane = jnp.int32(1) - jnp.minimum(lane, jnp.int32(1))
    flag = row_start.astype(jnp.int32)
    for i in range(_NSLICE):
        cur = tok_v[pl.ds(_L + i * _L, _L)]
        prev = tok_v[pl.ds(_L - 1 + i * _L, _L)]
        mixed = jnp.bitwise_xor(jnp.int32(36313) * cur, jnp.int32(27191) * prev)
        h = lax.rem(mixed, jnp.int32(_BIGRAM_VOCAB - 1))
        if i == 0:
            h = h + first_lane * flag * (jnp.int32(_BIGRAM_VOCAB - 1) - h)
        d = h - jnp.int32(_HALF)
        ge = jnp.int32(1) - lax.shift_right_logical(d, jnp.int32(31))
        j = jnp.int32(2) * (h - jnp.int32(_HALF) * ge) + ge
        idx_v[i // (_GB // _L), pl.ds((i % (_GB // _L)) * _L, _L)] = j

    pltpu.sync_copy(idx_v, idx_hbm.at[wid])


def _gather_body(idx_hbm, table_hbm, gat_hbm, idx_v, rows_v, sem):
    wid = lax.axis_index("s") * _NC + lax.axis_index("c")
    base = wid * _CHUNK
    pltpu.sync_copy(idx_hbm.at[wid], idx_v)

    copies = []
    for j in range(_NG):
        cp = pltpu.make_async_copy(
            table_hbm.at[idx_v.at[j]], rows_v.at[pl.ds(j * _GB, _GB)], sem
        )
        cp.start()
        copies.append(cp)
    for cp in copies:
        cp.wait()

    pltpu.sync_copy(rows_v, gat_hbm.at[pl.ds(base, _CHUNK), pl.ds(0, _BIGRAM_DIM)])


_sc_hash = pl.kernel(
    _hash_body,
    out_type=jax.ShapeDtypeStruct((_NW, _NG, _GB), jnp.int32),
    name="sc_hash",
    mesh=plsc.VectorSubcoreMesh(core_axis_name="c", subcore_axis_name="s"),
    scratch_types=[
        pltpu.VMEM((_L + _CHUNK,), jnp.int32),
        pltpu.VMEM((_NG, _GB), jnp.int32),
    ],
    compiler_params=pltpu.CompilerParams(use_tc_tiling_on_sc=False),
)

_sc_gather = pl.kernel(
    _gather_body,
    out_type=jax.ShapeDtypeStruct((_NTOK, 2 * _BIGRAM_DIM), jnp.float32),
    name="sc_gather",
    mesh=plsc.VectorSubcoreMesh(core_axis_name="c", subcore_axis_name="s"),
    scratch_types=[
        pltpu.VMEM((_NG, _GB), jnp.int32),
        pltpu.VMEM((_CHUNK, _BIGRAM_DIM), jnp.float32),
        pltpu.SemaphoreType.DMA,
    ],
    compiler_params=pltpu.CompilerParams(use_tc_tiling_on_sc=False),
)

_RB = 4096


def _mm_body(s_ref, x_ref, w_ref, o_ref):
    xa = x_ref[:, : _BIGRAM_DIM]
    acc = lax.dot_general(
        xa,
        w_ref[...],
        dimension_numbers=(((1,), (1,)), ((), ())),
        preferred_element_type=jnp.float32,
    )
    o_ref[...] = acc * s_ref[0]


def _tc_proj(x2, w, scale):
    return pl.pallas_call(
        _mm_body,
        grid=(_NTOK // _RB,),
        in_specs=[
            pl.BlockSpec(memory_space=pltpu.SMEM),
            pl.BlockSpec((_RB, 2 * _BIGRAM_DIM), lambda i: (i, 0)),
            pl.BlockSpec((_MODEL_DIM, _BIGRAM_DIM), lambda i: (0, 0)),
        ],
        out_specs=pl.BlockSpec((_RB, _MODEL_DIM), lambda i: (i, 0)),
        out_shape=jax.ShapeDtypeStruct((_NTOK, _MODEL_DIM), jnp.float32),
    )(scale.reshape(1), x2, w)


def kernel(token_ids, embed_weight, proj_weight, scale):
    tok = token_ids.astype(jnp.int32).reshape(_NTOK)
    idx = _sc_hash(tok)
    pairs = _pack(embed_weight.T)
    lin = pairs.reshape(_LINROWS, _BIGRAM_DIM)
    gathered = _sc_gather(idx, lin)
    out = _tc_proj(gathered, proj_weight, scale)
    return out.reshape(_BATCH, _SEQ, _MODEL_DIM)
